# baseline (speedup 1.0000x reference)
import jax
import jax.numpy as jnp
from jax import lax
from jax.experimental import pallas as pl
from jax.experimental.pallas import tpu as pltpu
from jax.experimental.pallas import tpu_sc as plsc

_NUM_RELATIONS = 1000
_DIM = 64
_CD = 32
_PD = 128
_BATCH = 4096
_NNEG = 32
_NC = 2
_NS = 16
_NW = _NC * _NS
_RPW = _BATCH // _NW
_L = 16
_CROWS = 128
_NCH = _RPW * _NNEG // _CROWS
_IR = _BATCH * _NNEG // 128


def _isqrt(m):
    im = plsc.bitcast(m, jnp.int32)
    y = plsc.bitcast(jnp.int32(0x5F3759DF) - (im >> 1), jnp.float32)
    t = (m * 0.5) * y
    y = y * (1.5 - t * y)
    return m * y


def _trig_body(ph_ref, out_ref):
    p = ph_ref[...]
    z = jnp.zeros_like(p)
    out_ref[...] = jnp.concatenate([jnp.cos(p), jnp.sin(p), z, z], axis=1)


_TPC = 2048


def _tp_body(x_ref, o_ref):
    xb = x_ref[...].astype(jnp.bfloat16)
    xt = jnp.swapaxes(xb, 0, 1).astype(jnp.float32)
    o_ref[:, pl.ds(0, _DIM)] = xt[0:_TPC // 2]
    o_ref[:, pl.ds(_DIM, _DIM)] = xt[_TPC // 2:]


def _row4(ref, r):
    return (ref[r, pl.ds(0, _L)], ref[r, pl.ds(_L, _L)],
            ref[r, pl.ds(_CD, _L)], ref[r, pl.ds(_CD + _L, _L)])


def _mag16(dr0, di0, dr1, di1):
    return (_isqrt(dr0 * dr0 + di0 * di0) + _isqrt(dr1 * dr1 + di1 * di1))


def _sc_body(ent, rc, hid, rid, tid, nh, nt,
             pos_o, ns_o,
             hid_v, rid_v, tid_v, nhi_v, nti_v,
             hof_v, tof_v, nho_v, nto_v,
             hh_v, hu_v, rc_v, nbuf0, nbuf1, ns_v, pos_v, st_v,
             semA, sem0, sem1):
    wid = lax.axis_index("s") * _NC + lax.axis_index("c")
    base = wid * _RPW
    lanes = lax.iota(jnp.int32, _L)

    pltpu.sync_copy(hid.at[pl.ds(base, _RPW)], hid_v)
    pltpu.sync_copy(rid.at[pl.ds(base, _RPW)], rid_v)
    pltpu.sync_copy(tid.at[pl.ds(base, _RPW)], tid_v)
    pltpu.sync_copy(nh.at[pl.ds(wid * (_IR // _NW), _IR // _NW)], nhi_v)
    pltpu.sync_copy(nt.at[pl.ds(wid * (_IR // _NW), _IR // _NW)], nti_v)

    def packmap(v):
        row = ((v >> 11) << 10) | (v & 1023)
        off = ((v >> 10) & 1) << 6
        return row, off

    def split1(ids_ref, off_ref):
        for k in range(_RPW // _L):
            row, off = packmap(ids_ref[pl.ds(k * _L, _L)])
            off_ref[pl.ds(k * _L, _L)] = off
            ids_ref[pl.ds(k * _L, _L)] = row

    split1(hid_v, hof_v)
    split1(tid_v, tof_v)

    def split2(ids_ref, off_ref):
        def body(k, carry):
            for m in range(128 // _L):
                row, off = packmap(ids_ref[k, pl.ds(m * _L, _L)])
                off_ref[k, pl.ds(m * _L, _L)] = off
                ids_ref[k, pl.ds(m * _L, _L)] = row
            return carry
        lax.fori_loop(0, _IR // _NW, body, jnp.int32(0))

    split2(nhi_v, nho_v)
    split2(nti_v, nto_v)

    dh = pltpu.async_copy(ent.at[hid_v], hh_v, semA)
    dt = pltpu.async_copy(ent.at[tid_v], hu_v, semA)
    dr = pltpu.async_copy(rc.at[rid_v], rc_v, semA)
    pltpu.async_copy(ent.at[nhi_v.at[0]], nbuf0, sem0)
    dh.wait()
    dt.wait()
    dr.wait()

    def colsum16(rows16):
        acc = plsc.load_gather(
            st_v, [rows16 + lanes, jnp.zeros((_L,), jnp.int32)])
        for l in range(1, _L):
            acc = acc + plsc.load_gather(
                st_v, [rows16 + lanes, jnp.full((_L,), l, jnp.int32)])
        return acc

    for g in range(_RPW // _L):

        def pos_body(r16, carry, g=g):
            r = g * _L + r16
            rsplat = jnp.full((_L,), r, jnp.int32)
            hof = plsc.load_gather(hof_v, [rsplat]) + lanes
            tof = plsc.load_gather(tof_v, [rsplat]) + lanes
            ar0 = plsc.load_gather(hh_v, [rsplat, hof])
            ar1 = plsc.load_gather(hh_v, [rsplat, hof + _L])
            ai0 = plsc.load_gather(hh_v, [rsplat, hof + _CD])
            ai1 = plsc.load_gather(hh_v, [rsplat, hof + (_CD + _L)])
            tr0 = plsc.load_gather(hu_v, [rsplat, tof])
            tr1 = plsc.load_gather(hu_v, [rsplat, tof + _L])
            ti0 = plsc.load_gather(hu_v, [rsplat, tof + _CD])
            ti1 = plsc.load_gather(hu_v, [rsplat, tof + (_CD + _L)])
            c0, c1, s0, s1 = _row4(rc_v, r)
            br0 = ar0 * c0 - ai0 * s0
            bi0 = ar0 * s0 + ai0 * c0
            br1 = ar1 * c1 - ai1 * s1
            bi1 = ar1 * s1 + ai1 * c1
            ur0 = tr0 * c0 + ti0 * s0
            ui0 = ti0 * c0 - tr0 * s0
            ur1 = tr1 * c1 + ti1 * s1
            ui1 = ti1 * c1 - tr1 * s1
            hh_v[r, pl.ds(0, _L)] = br0
            hh_v[r, pl.ds(_L, _L)] = br1
            hh_v[r, pl.ds(_CD, _L)] = bi0
            hh_v[r, pl.ds(_CD + _L, _L)] = bi1
            hu_v[r, pl.ds(0, _L)] = ur0
            hu_v[r, pl.ds(_L, _L)] = ur1
            hu_v[r, pl.ds(_CD, _L)] = ui0
            hu_v[r, pl.ds(_CD + _L, _L)] = ui1
            st_v[r16, pl.ds(0, _L)] = _mag16(ar0 - ur0, ai0 - ui0,
                                             ar1 - ur1, ai1 - ui1)
            return carry

        lax.fori_loop(0, _L, pos_body, jnp.int32(0), unroll=4)
        pos_v[pl.ds(g * _L, _L)] = -colsum16(0)
    pltpu.sync_copy(pos_v, pos_o.at[pl.ds(base, _RPW)])

    def neg_stage(idx2, off2, uref, colbase):
        def wait_fill(buf, sem):
            pltpu.make_async_copy(ent.at[pl.ds(0, _CROWS)], buf, sem).wait()

        def compute_chunk(c, buf):
            csplat = jnp.full((_L,), c, jnp.int32)
            for rl in range(_CROWS // _NNEG):
                r = c * (_CROWS // _NNEG) + rl
                ur0, ur1, ui0, ui1 = _row4(uref, r)

                def nb(n, carry, rl=rl, ur0=ur0, ur1=ur1, ui0=ui0, ui1=ui1,
                       csplat=csplat):
                    row = rl * _NNEG + n
                    rowsplat = jnp.full((_L,), row, jnp.int32)
                    off = plsc.load_gather(off2, [csplat, rowsplat]) + lanes
                    nr0 = plsc.load_gather(buf, [rowsplat, off])
                    nr1 = plsc.load_gather(buf, [rowsplat, off + _L])
                    ni0 = plsc.load_gather(buf, [rowsplat, off + _CD])
                    ni1 = plsc.load_gather(buf, [rowsplat, off + (_CD + _L)])
                    st_v[n, pl.ds(0, _L)] = _mag16(nr0 - ur0, ni0 - ui0,
                                                   nr1 - ur1, ni1 - ui1)
                    return carry

                lax.fori_loop(0, _NNEG, nb, jnp.int32(0), unroll=8)
                rsplat = jnp.full((_L,), r, jnp.int32)
                for grp in range(2):
                    plsc.store_scatter(
                        ns_v, [rsplat, lanes + (colbase + grp * _L)],
                        -colsum16(grp * _L))

        def pair_body(cp, carry):
            c0 = cp * 2
            pltpu.async_copy(ent.at[idx2.at[c0 + 1]], nbuf1, sem1)
            wait_fill(nbuf0, sem0)
            compute_chunk(c0, nbuf0)

            @pl.when(cp + 1 < _NCH // 2)
            def _():
                pltpu.async_copy(ent.at[idx2.at[c0 + 2]], nbuf0, sem0)

            wait_fill(nbuf1, sem1)
            compute_chunk(c0 + 1, nbuf1)
            return carry

        lax.fori_loop(0, _NCH // 2, pair_body, jnp.int32(0))

    neg_stage(nhi_v, nho_v, hu_v, 0)
    pltpu.async_copy(ent.at[nti_v.at[0]], nbuf0, sem0)
    neg_stage(nti_v, nto_v, hh_v, _NNEG)
    pltpu.sync_copy(ns_v, ns_o.at[pl.ds(base, _RPW)])


_sc_call = pl.kernel(
    _sc_body,
    out_type=(
        jax.ShapeDtypeStruct((_BATCH,), jnp.float32),
        jax.ShapeDtypeStruct((_BATCH, _PD), jnp.float32),
    ),
    mesh=plsc.VectorSubcoreMesh(core_axis_name="c", subcore_axis_name="s"),
    compiler_params=pltpu.CompilerParams(
        needs_layout_passes=False, use_tc_tiling_on_sc=True,
        disable_bounds_checks=True),
    scratch_types=[
        pltpu.VMEM((_RPW,), jnp.int32),
        pltpu.VMEM((_RPW,), jnp.int32),
        pltpu.VMEM((_RPW,), jnp.int32),
        pltpu.VMEM((_IR // _NW, 128), jnp.int32),
        pltpu.VMEM((_IR // _NW, 128), jnp.int32),
        pltpu.VMEM((_RPW,), jnp.int32),
        pltpu.VMEM((_RPW,), jnp.int32),
        pltpu.VMEM((_IR // _NW, 128), jnp.int32),
        pltpu.VMEM((_IR // _NW, 128), jnp.int32),
        pltpu.VMEM((_RPW, _PD), jnp.float32),
        pltpu.VMEM((_RPW, _PD), jnp.float32),
        pltpu.VMEM((_RPW, _PD), jnp.float32),
        pltpu.VMEM((_CROWS, _PD), jnp.float32),
        pltpu.VMEM((_CROWS, _PD), jnp.float32),
        pltpu.VMEM((_RPW, _PD), jnp.float32),
        pltpu.VMEM((_RPW,), jnp.float32),
        pltpu.VMEM((_NNEG, _L), jnp.float32),
        pltpu.SemaphoreType.DMA,
        pltpu.SemaphoreType.DMA,
        pltpu.SemaphoreType.DMA,
    ],
)


@jax.jit
def kernel(head_ids, relation_ids, tail_ids, negative_head_ids,
           negative_tail_ids, entity_embedding, relation_embedding):
    rc_tab = pl.pallas_call(
        _trig_body,
        out_shape=jax.ShapeDtypeStruct((_NUM_RELATIONS, _PD), jnp.float32),
    )(relation_embedding)
    nent = entity_embedding.shape[0]
    nblk = -(-nent // _TPC)
    ent_pad = pl.pallas_call(
        _tp_body,
        grid=(nblk,),
        in_specs=[pl.BlockSpec((_DIM, _TPC), lambda i: (0, i))],
        out_specs=pl.BlockSpec((_TPC // 2, _PD), lambda i: (i, 0)),
        out_shape=jax.ShapeDtypeStruct((nblk * _TPC // 2, _PD), jnp.float32),
    )(entity_embedding.T)
    nh2 = negative_head_ids.astype(jnp.int32).reshape(_IR, 128)
    nt2 = negative_tail_ids.astype(jnp.int32).reshape(_IR, 128)
    pos, ns = _sc_call(
        ent_pad, rc_tab,
        head_ids.astype(jnp.int32), relation_ids.astype(jnp.int32),
        tail_ids.astype(jnp.int32), nh2, nt2)
    return pos, ns[:, :2 * _NNEG]


# speedup vs baseline: 1.0153x; 1.0153x over previous
import jax
import jax.numpy as jnp
from jax import lax
from jax.experimental import pallas as pl
from jax.experimental.pallas import tpu as pltpu
from jax.experimental.pallas import tpu_sc as plsc

_NUM_RELATIONS = 1000
_DIM = 64
_CD = 32
_PD = 128
_BATCH = 4096
_NNEG = 32
_NC = 2
_NS = 16
_NW = _NC * _NS
_RPW = _BATCH // _NW
_L = 16
_CROWS = 128
_NCH = _RPW * _NNEG // _CROWS
_IR = _BATCH * _NNEG // 128


def _isqrt(m):
    im = plsc.bitcast(m, jnp.int32)
    y = plsc.bitcast(jnp.int32(0x5F3759DF) - (im >> 1), jnp.float32)
    t = (m * 0.5) * y
    y = y * (1.5 - t * y)
    return m * y


def _trig_body(ph_ref, out_ref):
    p = ph_ref[...]
    z = jnp.zeros_like(p)
    out_ref[...] = jnp.concatenate([jnp.cos(p), jnp.sin(p), z, z], axis=1)


_TPC = 2048


def _tp_body(x_ref, o_ref):
    xb = x_ref[...].astype(jnp.bfloat16)
    xt = jnp.swapaxes(xb, 0, 1).astype(jnp.float32)
    o_ref[:, pl.ds(0, _DIM)] = xt[0:_TPC // 2]
    o_ref[:, pl.ds(_DIM, _DIM)] = xt[_TPC // 2:]


def _row4(ref, r):
    return (ref[r, pl.ds(0, _L)], ref[r, pl.ds(_L, _L)],
            ref[r, pl.ds(_CD, _L)], ref[r, pl.ds(_CD + _L, _L)])


def _mag16(dr0, di0, dr1, di1):
    return (_isqrt(dr0 * dr0 + di0 * di0) + _isqrt(dr1 * dr1 + di1 * di1))


def _sc_body(ent, rc, hid, rid, tid, nh, nt,
             pos_o, ns_o,
             hid_v, rid_v, tid_v, nhi_v, nti_v,
             hof_v, tof_v, nho_v, nto_v,
             hh_v, hu_v, rc_v, nbuf0, nbuf1, ns_v, pos_v, st_v,
             semA, sem0, sem1):
    wid = lax.axis_index("s") * _NC + lax.axis_index("c")
    base = wid * _RPW
    lanes = lax.iota(jnp.int32, _L)

    pltpu.sync_copy(hid.at[pl.ds(base, _RPW)], hid_v)
    pltpu.sync_copy(rid.at[pl.ds(base, _RPW)], rid_v)
    pltpu.sync_copy(tid.at[pl.ds(base, _RPW)], tid_v)
    pltpu.sync_copy(nh.at[pl.ds(wid * (_IR // _NW), _IR // _NW)], nhi_v)
    pltpu.sync_copy(nt.at[pl.ds(wid * (_IR // _NW), _IR // _NW)], nti_v)

    def packmap(v):
        row = ((v >> 11) << 10) | (v & 1023)
        off = ((v >> 10) & 1) << 6
        return row, off

    def split1(ids_ref, off_ref):
        for k in range(_RPW // _L):
            row, off = packmap(ids_ref[pl.ds(k * _L, _L)])
            off_ref[pl.ds(k * _L, _L)] = off
            ids_ref[pl.ds(k * _L, _L)] = row

    split1(hid_v, hof_v)
    split1(tid_v, tof_v)

    def split2(ids_ref, off_ref):
        def body(k, carry):
            for m in range(128 // _L):
                row, off = packmap(ids_ref[k, pl.ds(m * _L, _L)])
                off_ref[k, pl.ds(m * _L, _L)] = off
                ids_ref[k, pl.ds(m * _L, _L)] = row
            return carry
        lax.fori_loop(0, _IR // _NW, body, jnp.int32(0))

    split2(nhi_v, nho_v)
    split2(nti_v, nto_v)

    dh = pltpu.async_copy(ent.at[hid_v], hh_v, semA)
    dt = pltpu.async_copy(ent.at[tid_v], hu_v, semA)
    dr = pltpu.async_copy(rc.at[rid_v], rc_v, semA)
    pltpu.async_copy(ent.at[nhi_v.at[0]], nbuf0, sem0)
    dh.wait()
    dt.wait()
    dr.wait()

    def colsum16(rows16):
        acc = plsc.load_gather(
            st_v, [rows16 + lanes, jnp.zeros((_L,), jnp.int32)])
        for l in range(1, _L):
            acc = acc + plsc.load_gather(
                st_v, [rows16 + lanes, jnp.full((_L,), l, jnp.int32)])
        return acc

    for g in range(_RPW // _L):

        def pos_body(r16, carry, g=g):
            r = g * _L + r16
            rsplat = jnp.full((_L,), r, jnp.int32)
            hof = plsc.load_gather(hof_v, [rsplat]) + lanes
            tof = plsc.load_gather(tof_v, [rsplat]) + lanes
            ar0 = plsc.load_gather(hh_v, [rsplat, hof])
            ar1 = plsc.load_gather(hh_v, [rsplat, hof + _L])
            ai0 = plsc.load_gather(hh_v, [rsplat, hof + _CD])
            ai1 = plsc.load_gather(hh_v, [rsplat, hof + (_CD + _L)])
            tr0 = plsc.load_gather(hu_v, [rsplat, tof])
            tr1 = plsc.load_gather(hu_v, [rsplat, tof + _L])
            ti0 = plsc.load_gather(hu_v, [rsplat, tof + _CD])
            ti1 = plsc.load_gather(hu_v, [rsplat, tof + (_CD + _L)])
            c0, c1, s0, s1 = _row4(rc_v, r)
            br0 = ar0 * c0 - ai0 * s0
            bi0 = ar0 * s0 + ai0 * c0
            br1 = ar1 * c1 - ai1 * s1
            bi1 = ar1 * s1 + ai1 * c1
            ur0 = tr0 * c0 + ti0 * s0
            ui0 = ti0 * c0 - tr0 * s0
            ur1 = tr1 * c1 + ti1 * s1
            ui1 = ti1 * c1 - tr1 * s1
            hh_v[r, pl.ds(0, _L)] = br0
            hh_v[r, pl.ds(_L, _L)] = br1
            hh_v[r, pl.ds(_CD, _L)] = bi0
            hh_v[r, pl.ds(_CD + _L, _L)] = bi1
            hu_v[r, pl.ds(0, _L)] = ur0
            hu_v[r, pl.ds(_L, _L)] = ur1
            hu_v[r, pl.ds(_CD, _L)] = ui0
            hu_v[r, pl.ds(_CD + _L, _L)] = ui1
            st_v[r16, pl.ds(0, _L)] = _mag16(ar0 - ur0, ai0 - ui0,
                                             ar1 - ur1, ai1 - ui1)
            return carry

        lax.fori_loop(0, _L, pos_body, jnp.int32(0), unroll=2)
        pos_v[pl.ds(g * _L, _L)] = -colsum16(0)
    pltpu.sync_copy(pos_v, pos_o.at[pl.ds(base, _RPW)])

    def neg_stage(idx2, off2, uref, colbase):
        def wait_fill(buf, sem):
            pltpu.make_async_copy(ent.at[pl.ds(0, _CROWS)], buf, sem).wait()

        def compute_chunk(c, buf):
            csplat = jnp.full((_L,), c, jnp.int32)
            for rl in range(_CROWS // _NNEG):
                r = c * (_CROWS // _NNEG) + rl
                ur0, ur1, ui0, ui1 = _row4(uref, r)

                def nb(n, carry, rl=rl, ur0=ur0, ur1=ur1, ui0=ui0, ui1=ui1,
                       csplat=csplat):
                    row = rl * _NNEG + n
                    rowsplat = jnp.full((_L,), row, jnp.int32)
                    off = plsc.load_gather(off2, [csplat, rowsplat]) + lanes
                    nr0 = plsc.load_gather(buf, [rowsplat, off])
                    nr1 = plsc.load_gather(buf, [rowsplat, off + _L])
                    ni0 = plsc.load_gather(buf, [rowsplat, off + _CD])
                    ni1 = plsc.load_gather(buf, [rowsplat, off + (_CD + _L)])
                    st_v[n, pl.ds(0, _L)] = _mag16(nr0 - ur0, ni0 - ui0,
                                                   nr1 - ur1, ni1 - ui1)
                    return carry

                lax.fori_loop(0, _NNEG, nb, jnp.int32(0), unroll=4)
                rsplat = jnp.full((_L,), r, jnp.int32)
                for grp in range(2):
                    plsc.store_scatter(
                        ns_v, [rsplat, lanes + (colbase + grp * _L)],
                        -colsum16(grp * _L))

        def pair_body(cp, carry):
            c0 = cp * 2
            pltpu.async_copy(ent.at[idx2.at[c0 + 1]], nbuf1, sem1)
            wait_fill(nbuf0, sem0)
            compute_chunk(c0, nbuf0)

            @pl.when(cp + 1 < _NCH // 2)
            def _():
                pltpu.async_copy(ent.at[idx2.at[c0 + 2]], nbuf0, sem0)

            wait_fill(nbuf1, sem1)
            compute_chunk(c0 + 1, nbuf1)
            return carry

        lax.fori_loop(0, _NCH // 2, pair_body, jnp.int32(0))

    neg_stage(nhi_v, nho_v, hu_v, 0)
    pltpu.async_copy(ent.at[nti_v.at[0]], nbuf0, sem0)
    neg_stage(nti_v, nto_v, hh_v, _NNEG)
    pltpu.sync_copy(ns_v, ns_o.at[pl.ds(base, _RPW)])


_sc_call = pl.kernel(
    _sc_body,
    out_type=(
        jax.ShapeDtypeStruct((_BATCH,), jnp.float32),
        jax.ShapeDtypeStruct((_BATCH, _PD), jnp.float32),
    ),
    mesh=plsc.VectorSubcoreMesh(core_axis_name="c", subcore_axis_name="s"),
    compiler_params=pltpu.CompilerParams(
        needs_layout_passes=False, use_tc_tiling_on_sc=True,
        disable_bounds_checks=True),
    scratch_types=[
        pltpu.VMEM((_RPW,), jnp.int32),
        pltpu.VMEM((_RPW,), jnp.int32),
        pltpu.VMEM((_RPW,), jnp.int32),
        pltpu.VMEM((_IR // _NW, 128), jnp.int32),
        pltpu.VMEM((_IR // _NW, 128), jnp.int32),
        pltpu.VMEM((_RPW,), jnp.int32),
        pltpu.VMEM((_RPW,), jnp.int32),
        pltpu.VMEM((_IR // _NW, 128), jnp.int32),
        pltpu.VMEM((_IR // _NW, 128), jnp.int32),
        pltpu.VMEM((_RPW, _PD), jnp.float32),
        pltpu.VMEM((_RPW, _PD), jnp.float32),
        pltpu.VMEM((_RPW, _PD), jnp.float32),
        pltpu.VMEM((_CROWS, _PD), jnp.float32),
        pltpu.VMEM((_CROWS, _PD), jnp.float32),
        pltpu.VMEM((_RPW, _PD), jnp.float32),
        pltpu.VMEM((_RPW,), jnp.float32),
        pltpu.VMEM((_NNEG, _L), jnp.float32),
        pltpu.SemaphoreType.DMA,
        pltpu.SemaphoreType.DMA,
        pltpu.SemaphoreType.DMA,
    ],
)


@jax.jit
def kernel(head_ids, relation_ids, tail_ids, negative_head_ids,
           negative_tail_ids, entity_embedding, relation_embedding):
    rc_tab = pl.pallas_call(
        _trig_body,
        out_shape=jax.ShapeDtypeStruct((_NUM_RELATIONS, _PD), jnp.float32),
    )(relation_embedding)
    nent = entity_embedding.shape[0]
    nblk = -(-nent // _TPC)
    ent_pad = pl.pallas_call(
        _tp_body,
        grid=(nblk,),
        in_specs=[pl.BlockSpec((_DIM, _TPC), lambda i: (0, i))],
        out_specs=pl.BlockSpec((_TPC // 2, _PD), lambda i: (i, 0)),
        out_shape=jax.ShapeDtypeStruct((nblk * _TPC // 2, _PD), jnp.float32),
    )(entity_embedding.T)
    nh2 = negative_head_ids.astype(jnp.int32).reshape(_IR, 128)
    nt2 = negative_tail_ids.astype(jnp.int32).reshape(_IR, 128)
    pos, ns = _sc_call(
        ent_pad, rc_tab,
        head_ids.astype(jnp.int32), relation_ids.astype(jnp.int32),
        tail_ids.astype(jnp.int32), nh2, nt2)
    return pos, ns[:, :2 * _NNEG]
